# Initial kernel scaffold; baseline (speedup 1.0000x reference)
import jax
import jax.numpy as jnp
from jax.experimental import pallas as pl


def kernel(x, edge_index, batch, edge_attr, Wc1, bc1, Wc2, bc2, Wc3, bc3, W1, b1, W2, b2):
    raise NotImplementedError("write your pallas kernel here")


# speedup vs baseline: 6.5460x; 6.5460x over previous
import functools

import jax
import jax.numpy as jnp
from jax import lax
from jax.experimental import pallas as pl
from jax.experimental.pallas import tpu as pltpu
from jax.experimental.pallas import tpu_sc as plsc

N_NODES = 10000
D = 128
MAX_NODES = 90
NB = (N_NODES - 1) // MAX_NODES + 1
E_RAW = 320000
E_ALL = E_RAW + N_NODES
CHUNK = 128
N_TILES = 32
CHUNKS_PER_TILE = -(-E_ALL // (N_TILES * CHUNK))
EPT = CHUNKS_PER_TILE * CHUNK
E_PAD = N_TILES * EPT
DEG_ROWS = 80
STRIPE = 624

_HIGH = lax.Precision.HIGHEST

_mesh = plsc.VectorSubcoreMesh(core_axis_name="c", subcore_axis_name="s")


_DEG_STRIPE = DEG_ROWS * D // 16


def _sc_deg_body(dst_hbm, ew_hbm, out_hbm, dacc, zb, dst_v, ew_v):
    c = lax.axis_index("c")
    s = lax.axis_index("s")
    zv = jnp.zeros((16,), jnp.float32)

    def zrow(r, carry):
        zb[pl.ds(r * 16, 16)] = zv
        return carry

    lax.fori_loop(0, _DEG_STRIPE // 16, zrow, 0)
    base = s * _DEG_STRIPE
    pltpu.sync_copy(zb, dacc.at[pl.ds(base, _DEG_STRIPE)])
    plsc.subcore_barrier()

    ebase = (c * 16 + s) * EPT

    def chunk_body(i, carry):
        off = ebase + i * CHUNK
        pltpu.sync_copy(dst_hbm.at[pl.ds(off, CHUNK)], dst_v)
        pltpu.sync_copy(ew_hbm.at[pl.ds(off, CHUNK)], ew_v)
        pltpu.sync_copy(ew_v, dacc.at[dst_v], add=True)
        return carry

    lax.fori_loop(0, CHUNKS_PER_TILE, chunk_body, 0)
    plsc.subcore_barrier()
    pltpu.sync_copy(dacc.at[pl.ds(base, _DEG_STRIPE)],
                    out_hbm.at[c, pl.ds(base, _DEG_STRIPE)])


_sc_deg = pl.kernel(
    _sc_deg_body,
    out_type=jax.ShapeDtypeStruct((2, DEG_ROWS * D), jnp.float32),
    mesh=_mesh,
    scratch_types=[
        pltpu.VMEM_SHARED((DEG_ROWS * D,), jnp.float32),
        pltpu.VMEM((_DEG_STRIPE,), jnp.float32),
        pltpu.VMEM((CHUNK,), jnp.int32),
        pltpu.VMEM((CHUNK,), jnp.float32),
    ],
)


def _sc_scatter_body(g_hbm, src_hbm, dst_hbm, ewr_hbm, out_hbm,
                     acc, src_v, dst_v, ewr_v, rows, zbuf, sem):
    c = lax.axis_index("c")
    s = lax.axis_index("s")
    zv = jnp.zeros((16,), jnp.float32)

    def zrow(r, carry):
        for cc in range(8):
            zbuf[r, pl.ds(cc * 16, 16)] = zv
        return carry

    lax.fori_loop(0, CHUNK, zrow, 0)

    base = s * STRIPE
    for j in range(STRIPE // CHUNK):
        pltpu.sync_copy(zbuf, acc.at[pl.ds(base + j * CHUNK, CHUNK)])
    rem = STRIPE % CHUNK
    if rem:
        pltpu.sync_copy(zbuf.at[pl.ds(0, rem)],
                        acc.at[pl.ds(base + (STRIPE // CHUNK) * CHUNK, rem)])
    tail = N_NODES - 16 * STRIPE

    @pl.when(s == 15)
    def _zero_tail():
        pltpu.sync_copy(zbuf.at[pl.ds(0, tail)],
                        acc.at[pl.ds(16 * STRIPE, tail)])

    plsc.subcore_barrier()

    ebase = (c * 16 + s) * EPT

    def chunk_body(i, carry):
        off = ebase + i * CHUNK
        pltpu.sync_copy(src_hbm.at[pl.ds(off, CHUNK)], src_v)
        pltpu.sync_copy(dst_hbm.at[pl.ds(off, CHUNK)], dst_v)
        pltpu.sync_copy(ewr_hbm.at[pl.ds(off * 16, CHUNK * 16)], ewr_v)
        pltpu.async_copy(g_hbm.at[src_v], rows, sem).wait()

        def row_scale(r, inner):
            w16 = ewr_v[pl.ds(r * 16, 16)]
            for cc in range(8):
                rows[r, pl.ds(cc * 16, 16)] = rows[r, pl.ds(cc * 16, 16)] * w16
            return inner

        lax.fori_loop(0, CHUNK, row_scale, 0)
        pltpu.sync_copy(rows, acc.at[dst_v], add=True)
        return carry

    lax.fori_loop(0, CHUNKS_PER_TILE, chunk_body, 0)
    plsc.subcore_barrier()
    pltpu.sync_copy(acc.at[pl.ds(base, STRIPE)],
                    out_hbm.at[c, pl.ds(base, STRIPE)])

    @pl.when(s == 15)
    def _write_tail():
        pltpu.sync_copy(acc.at[pl.ds(16 * STRIPE, tail)],
                        out_hbm.at[c, pl.ds(16 * STRIPE, tail)])


_sc_scatter = pl.kernel(
    _sc_scatter_body,
    out_type=jax.ShapeDtypeStruct((2, N_NODES, D), jnp.float32),
    mesh=_mesh,
    scratch_types=[
        pltpu.VMEM_SHARED((N_NODES, D), jnp.float32),
        pltpu.VMEM((CHUNK,), jnp.int32),
        pltpu.VMEM((CHUNK,), jnp.int32),
        pltpu.VMEM((CHUNK * 16,), jnp.float32),
        pltpu.VMEM((CHUNK, D), jnp.float32),
        pltpu.VMEM((CHUNK, D), jnp.float32),
        pltpu.SemaphoreType.DMA,
    ],
)


def _dis_body(degp_ref, dis_ref):
    degsum = jnp.sum(degp_ref[...], axis=0)
    pos = degsum > 0
    dis_ref[...] = jnp.where(pos, lax.rsqrt(jnp.where(pos, degsum, 1.0)), 0.0)


def _tc_dis(degp):
    return pl.pallas_call(
        _dis_body,
        out_shape=jax.ShapeDtypeStruct((DEG_ROWS, D), jnp.float32),
    )(degp)


_ROWS_BLK = 1000
_GRID = N_NODES // _ROWS_BLK


def _in_body(x_ref, w_ref, dis_ref, g_ref):
    g_ref[...] = dis_ref[...] * jnp.dot(
        x_ref[...], w_ref[...], preferred_element_type=jnp.float32,
        precision=_HIGH)


def _tc_in(x, W, dis):
    return pl.pallas_call(
        _in_body,
        grid=(_GRID,),
        in_specs=[
            pl.BlockSpec((_ROWS_BLK, D), lambda i: (i, 0)),
            pl.BlockSpec((D, D), lambda i: (0, 0)),
            pl.BlockSpec((_ROWS_BLK, 1), lambda i: (i, 0)),
        ],
        out_specs=pl.BlockSpec((_ROWS_BLK, D), lambda i: (i, 0)),
        out_shape=jax.ShapeDtypeStruct((N_NODES, D), jnp.float32),
    )(x, W, dis)


def _mid_body(m_ref, b_ref, dis_ref, w_ref, h_ref, g_ref):
    acc = m_ref[0] + m_ref[1]
    h = jnp.maximum(dis_ref[...] * acc + b_ref[...], 0.0)
    h_ref[...] = h
    g_ref[...] = dis_ref[...] * jnp.dot(
        h, w_ref[...], preferred_element_type=jnp.float32, precision=_HIGH)


def _tc_mid(m, b2d, dis, W):
    return pl.pallas_call(
        _mid_body,
        grid=(_GRID,),
        in_specs=[
            pl.BlockSpec((2, _ROWS_BLK, D), lambda i: (0, i, 0)),
            pl.BlockSpec((1, D), lambda i: (0, 0)),
            pl.BlockSpec((_ROWS_BLK, 1), lambda i: (i, 0)),
            pl.BlockSpec((D, D), lambda i: (0, 0)),
        ],
        out_specs=[
            pl.BlockSpec((_ROWS_BLK, D), lambda i: (i, 0)),
            pl.BlockSpec((_ROWS_BLK, D), lambda i: (i, 0)),
        ],
        out_shape=[
            jax.ShapeDtypeStruct((N_NODES, D), jnp.float32),
            jax.ShapeDtypeStruct((N_NODES, D), jnp.float32),
        ],
    )(m, b2d, dis, W)


def _out_body(m_ref, b_ref, dis_ref, h_ref):
    acc = m_ref[0] + m_ref[1]
    h_ref[...] = jnp.maximum(dis_ref[...] * acc + b_ref[...], 0.0)


def _tc_out(m, b2d, dis):
    return pl.pallas_call(
        _out_body,
        grid=(_GRID,),
        in_specs=[
            pl.BlockSpec((2, _ROWS_BLK, D), lambda i: (0, i, 0)),
            pl.BlockSpec((1, D), lambda i: (0, 0)),
            pl.BlockSpec((_ROWS_BLK, 1), lambda i: (i, 0)),
        ],
        out_specs=pl.BlockSpec((_ROWS_BLK, D), lambda i: (i, 0)),
        out_shape=jax.ShapeDtypeStruct((N_NODES, D), jnp.float32),
    )(m, b2d, dis)


_N_PADDED = NB * MAX_NODES


def _pad_body(h1_ref, h2_ref, h3_ref, xp_ref):
    h1 = h1_ref[...]
    h2 = h2_ref[...]
    h3 = h3_ref[...]
    m = jnp.minimum(jnp.minimum(jnp.min(h1), jnp.min(h2)), jnp.min(h3))
    fill = m - 1.0
    xp_ref[pl.ds(0, N_NODES), pl.ds(0, D)] = h1
    xp_ref[pl.ds(0, N_NODES), pl.ds(D, D)] = h2
    xp_ref[pl.ds(0, N_NODES), pl.ds(2 * D, D)] = h3
    xp_ref[pl.ds(N_NODES, _N_PADDED - N_NODES), :] = jnp.full(
        (_N_PADDED - N_NODES, 3 * D), fill, jnp.float32)


def _tc_pad(h1, h2, h3):
    return pl.pallas_call(
        _pad_body,
        out_shape=jax.ShapeDtypeStruct((_N_PADDED, 3 * D), jnp.float32),
    )(h1, h2, h3)


def _mlp_body(xp_ref, w1_ref, b1_ref, w2_ref, b2_ref, out_ref):
    z = jnp.dot(xp_ref[...], w1_ref[...], preferred_element_type=jnp.float32,
                precision=_HIGH) + b1_ref[...]
    z = jnp.maximum(z, 0.0)
    z = jnp.dot(z, w2_ref[...], preferred_element_type=jnp.float32,
                precision=_HIGH) + b2_ref[...]
    mx = jnp.max(z, axis=-1, keepdims=True)
    lse = jnp.log(jnp.sum(jnp.exp(z - mx), axis=-1, keepdims=True)) + mx
    out_ref[...] = z - lse


def _tc_mlp(xp2, W1, b1_2d, W2, b2_2d):
    return pl.pallas_call(
        _mlp_body,
        out_shape=jax.ShapeDtypeStruct((NB, W2.shape[1]), jnp.float32),
    )(xp2, W1, b1_2d, W2, b2_2d)


@jax.jit
def kernel(x, edge_index, batch, edge_attr, Wc1, bc1, Wc2, bc2, Wc3, bc3,
           W1, b1, W2, b2):
    n = x.shape[0]
    loop_idx = jnp.arange(n, dtype=jnp.int32)
    pad = E_PAD - E_ALL
    src = jnp.concatenate([edge_index[0], loop_idx,
                           jnp.zeros((pad,), jnp.int32)])
    dst = jnp.concatenate([edge_index[1], loop_idx,
                           jnp.zeros((pad,), jnp.int32)])
    ew = jnp.concatenate([edge_attr, jnp.ones((n,), jnp.float32),
                          jnp.zeros((pad,), jnp.float32)])
    ewr = jnp.repeat(ew, 16)

    degp = _sc_deg(dst, ew)
    dis2d = _tc_dis(degp.reshape(2, DEG_ROWS, D))
    dis = dis2d.reshape(-1)[:N_NODES].reshape(N_NODES, 1)

    g1 = _tc_in(x, Wc1, dis)
    m1 = _sc_scatter(g1, src, dst, ewr)
    h1, g2 = _tc_mid(m1, bc1.reshape(1, -1), dis, Wc2)
    m2 = _sc_scatter(g2, src, dst, ewr)
    h2, g3 = _tc_mid(m2, bc2.reshape(1, -1), dis, Wc3)
    m3 = _sc_scatter(g3, src, dst, ewr)
    h3 = _tc_out(m3, bc3.reshape(1, -1), dis)

    xp = _tc_pad(h1, h2, h3)
    out = _tc_mlp(xp.reshape(NB, MAX_NODES * 3 * D), W1,
                  b1.reshape(1, -1), W2, b2.reshape(1, -1))
    return out


# speedup vs baseline: 8.1164x; 1.2399x over previous
import functools

import jax
import jax.numpy as jnp
from jax import lax
from jax.experimental import pallas as pl
from jax.experimental.pallas import tpu as pltpu
from jax.experimental.pallas import tpu_sc as plsc

N_NODES = 10000
D = 128
MAX_NODES = 90
NB = (N_NODES - 1) // MAX_NODES + 1
E_RAW = 320000
E_ALL = E_RAW + N_NODES
CHUNK = 128
N_TILES = 32
CHUNKS_PER_TILE = 82
EPT = CHUNKS_PER_TILE * CHUNK
E_PAD = N_TILES * EPT
DEG_ROWS = 80
STRIPE = 624

_HIGH = lax.Precision.HIGHEST

_mesh = plsc.VectorSubcoreMesh(core_axis_name="c", subcore_axis_name="s")


_DEG_STRIPE = DEG_ROWS * D // 16


def _sc_deg_body(dst_hbm, ew_hbm, out_hbm, dacc, zb, dst_v, ew_v):
    c = lax.axis_index("c")
    s = lax.axis_index("s")
    zv = jnp.zeros((16,), jnp.float32)

    def zrow(r, carry):
        zb[pl.ds(r * 16, 16)] = zv
        return carry

    lax.fori_loop(0, _DEG_STRIPE // 16, zrow, 0)
    base = s * _DEG_STRIPE
    pltpu.sync_copy(zb, dacc.at[pl.ds(base, _DEG_STRIPE)])
    plsc.subcore_barrier()

    ebase = (c * 16 + s) * EPT

    def chunk_body(i, carry):
        off = ebase + i * CHUNK
        pltpu.sync_copy(dst_hbm.at[pl.ds(off, CHUNK)], dst_v)
        pltpu.sync_copy(ew_hbm.at[pl.ds(off, CHUNK)], ew_v)
        pltpu.sync_copy(ew_v, dacc.at[dst_v], add=True)
        return carry

    lax.fori_loop(0, CHUNKS_PER_TILE, chunk_body, 0)
    plsc.subcore_barrier()
    pltpu.sync_copy(dacc.at[pl.ds(base, _DEG_STRIPE)],
                    out_hbm.at[c, pl.ds(base, _DEG_STRIPE)])


_sc_deg = pl.kernel(
    _sc_deg_body,
    out_type=jax.ShapeDtypeStruct((2, DEG_ROWS * D), jnp.float32),
    mesh=_mesh,
    scratch_types=[
        pltpu.VMEM_SHARED((DEG_ROWS * D,), jnp.float32),
        pltpu.VMEM((_DEG_STRIPE,), jnp.float32),
        pltpu.VMEM((CHUNK,), jnp.int32),
        pltpu.VMEM((CHUNK,), jnp.float32),
    ],
)


_ZROWS = 64


def _sc_scatter_body(g_hbm, src_hbm, dst_hbm, ewr_hbm, out_hbm,
                     acc, src0, src1, dst0, dst1, ewr0, ewr1, rows0, rows1,
                     zbuf, sg0, sg1, ss0, ss1, se0, se1, sd0, sd1):
    c = lax.axis_index("c")
    s = lax.axis_index("s")
    src_v = (src0, src1)
    dst_v = (dst0, dst1)
    ewr_v = (ewr0, ewr1)
    rows = (rows0, rows1)
    sem_g = (sg0, sg1)
    sem_src = (ss0, ss1)
    sem_ewr = (se0, se1)
    sem_dst = (sd0, sd1)
    zv = jnp.zeros((16,), jnp.float32)

    def zrow(r, carry):
        for cc in range(8):
            zbuf[r, pl.ds(cc * 16, 16)] = zv
        return carry

    lax.fori_loop(0, _ZROWS, zrow, 0)

    base = s * STRIPE
    for j in range(STRIPE // _ZROWS):
        pltpu.sync_copy(zbuf, acc.at[pl.ds(base + j * _ZROWS, _ZROWS)])
    rem = STRIPE % _ZROWS
    if rem:
        pltpu.sync_copy(zbuf.at[pl.ds(0, rem)],
                        acc.at[pl.ds(base + (STRIPE // _ZROWS) * _ZROWS, rem)])
    tail = N_NODES - 16 * STRIPE

    @pl.when(s == 15)
    def _zero_tail():
        pltpu.sync_copy(zbuf.at[pl.ds(0, tail)],
                        acc.at[pl.ds(16 * STRIPE, tail)])

    plsc.subcore_barrier()

    ebase = (c * 16 + s) * EPT

    def lds(i, b):
        off = ebase + i * CHUNK
        return (
            pltpu.make_async_copy(src_hbm.at[pl.ds(off, CHUNK)], src_v[b],
                                  sem_src[b]),
            pltpu.make_async_copy(ewr_hbm.at[pl.ds(off * 16, CHUNK * 16)],
                                  ewr_v[b], sem_ewr[b]),
            pltpu.make_async_copy(dst_hbm.at[pl.ds(off, CHUNK)], dst_v[b],
                                  sem_dst[b]),
        )

    def gat(b):
        return pltpu.make_async_copy(g_hbm.at[src_v[b]], rows[b], sem_g[b])

    for b in (0, 1):
        cs, ce, cd = lds(b, b)
        cs.start()
        ce.start()
        cd.start()
        cs.wait()
        gat(b).start()

    def process(i, b):
        nxt = i + 2
        have_nxt = nxt < CHUNKS_PER_TILE
        gat(b).wait()
        cs, ce, cd = lds(nxt, b)

        @pl.when(have_nxt)
        def _start_src():
            cs.start()

        ce.wait()

        def row_scale(r, inner):
            for u in range(2):
                rr = r * 2 + u
                w16 = ewr_v[b][pl.ds(rr * 16, 16)]
                for cc in range(8):
                    rows[b][rr, pl.ds(cc * 16, 16)] = (
                        rows[b][rr, pl.ds(cc * 16, 16)] * w16)
            return inner

        lax.fori_loop(0, CHUNK // 2, row_scale, 0)

        @pl.when(have_nxt)
        def _start_ewr():
            ce.start()

        cd.wait()
        pltpu.sync_copy(rows[b], acc.at[dst_v[b]], add=True)

        @pl.when(have_nxt)
        def _prefetch_rest():
            cd.start()
            cs.wait()
            gat(b).start()

    def pair(j, carry):
        process(2 * j, 0)
        process(2 * j + 1, 1)
        return carry

    lax.fori_loop(0, CHUNKS_PER_TILE // 2, pair, 0)
    plsc.subcore_barrier()
    pltpu.sync_copy(acc.at[pl.ds(base, STRIPE)],
                    out_hbm.at[c, pl.ds(base, STRIPE)])

    @pl.when(s == 15)
    def _write_tail():
        pltpu.sync_copy(acc.at[pl.ds(16 * STRIPE, tail)],
                        out_hbm.at[c, pl.ds(16 * STRIPE, tail)])


_sc_scatter = pl.kernel(
    _sc_scatter_body,
    out_type=jax.ShapeDtypeStruct((2, N_NODES, D), jnp.float32),
    mesh=_mesh,
    scratch_types=[
        pltpu.VMEM_SHARED((N_NODES, D), jnp.float32),
        pltpu.VMEM((CHUNK,), jnp.int32),
        pltpu.VMEM((CHUNK,), jnp.int32),
        pltpu.VMEM((CHUNK,), jnp.int32),
        pltpu.VMEM((CHUNK,), jnp.int32),
        pltpu.VMEM((CHUNK * 16,), jnp.float32),
        pltpu.VMEM((CHUNK * 16,), jnp.float32),
        pltpu.VMEM((CHUNK, D), jnp.float32),
        pltpu.VMEM((CHUNK, D), jnp.float32),
        pltpu.VMEM((_ZROWS, D), jnp.float32),
        pltpu.SemaphoreType.DMA,
        pltpu.SemaphoreType.DMA,
        pltpu.SemaphoreType.DMA,
        pltpu.SemaphoreType.DMA,
        pltpu.SemaphoreType.DMA,
        pltpu.SemaphoreType.DMA,
        pltpu.SemaphoreType.DMA,
        pltpu.SemaphoreType.DMA,
    ],
)


def _dis_body(degp_ref, dis_ref):
    degsum = jnp.sum(degp_ref[...], axis=0)
    pos = degsum > 0
    dis_ref[...] = jnp.where(pos, lax.rsqrt(jnp.where(pos, degsum, 1.0)), 0.0)


def _tc_dis(degp):
    return pl.pallas_call(
        _dis_body,
        out_shape=jax.ShapeDtypeStruct((DEG_ROWS, D), jnp.float32),
    )(degp)


_ROWS_BLK = 1000
_GRID = N_NODES // _ROWS_BLK


def _in_body(x_ref, w_ref, dis_ref, g_ref):
    g_ref[...] = dis_ref[...] * jnp.dot(
        x_ref[...], w_ref[...], preferred_element_type=jnp.float32,
        precision=_HIGH)


def _tc_in(x, W, dis):
    return pl.pallas_call(
        _in_body,
        grid=(_GRID,),
        in_specs=[
            pl.BlockSpec((_ROWS_BLK, D), lambda i: (i, 0)),
            pl.BlockSpec((D, D), lambda i: (0, 0)),
            pl.BlockSpec((_ROWS_BLK, 1), lambda i: (i, 0)),
        ],
        out_specs=pl.BlockSpec((_ROWS_BLK, D), lambda i: (i, 0)),
        out_shape=jax.ShapeDtypeStruct((N_NODES, D), jnp.float32),
    )(x, W, dis)


def _mid_body(m_ref, b_ref, dis_ref, w_ref, h_ref, g_ref):
    acc = m_ref[0] + m_ref[1]
    h = jnp.maximum(dis_ref[...] * acc + b_ref[...], 0.0)
    h_ref[...] = h
    g_ref[...] = dis_ref[...] * jnp.dot(
        h, w_ref[...], preferred_element_type=jnp.float32, precision=_HIGH)


def _tc_mid(m, b2d, dis, W):
    return pl.pallas_call(
        _mid_body,
        grid=(_GRID,),
        in_specs=[
            pl.BlockSpec((2, _ROWS_BLK, D), lambda i: (0, i, 0)),
            pl.BlockSpec((1, D), lambda i: (0, 0)),
            pl.BlockSpec((_ROWS_BLK, 1), lambda i: (i, 0)),
            pl.BlockSpec((D, D), lambda i: (0, 0)),
        ],
        out_specs=[
            pl.BlockSpec((_ROWS_BLK, D), lambda i: (i, 0)),
            pl.BlockSpec((_ROWS_BLK, D), lambda i: (i, 0)),
        ],
        out_shape=[
            jax.ShapeDtypeStruct((N_NODES, D), jnp.float32),
            jax.ShapeDtypeStruct((N_NODES, D), jnp.float32),
        ],
    )(m, b2d, dis, W)


def _out_body(m_ref, b_ref, dis_ref, h_ref):
    acc = m_ref[0] + m_ref[1]
    h_ref[...] = jnp.maximum(dis_ref[...] * acc + b_ref[...], 0.0)


def _tc_out(m, b2d, dis):
    return pl.pallas_call(
        _out_body,
        grid=(_GRID,),
        in_specs=[
            pl.BlockSpec((2, _ROWS_BLK, D), lambda i: (0, i, 0)),
            pl.BlockSpec((1, D), lambda i: (0, 0)),
            pl.BlockSpec((_ROWS_BLK, 1), lambda i: (i, 0)),
        ],
        out_specs=pl.BlockSpec((_ROWS_BLK, D), lambda i: (i, 0)),
        out_shape=jax.ShapeDtypeStruct((N_NODES, D), jnp.float32),
    )(m, b2d, dis)


_N_PADDED = NB * MAX_NODES


def _pad_body(h1_ref, h2_ref, h3_ref, xp_ref):
    h1 = h1_ref[...]
    h2 = h2_ref[...]
    h3 = h3_ref[...]
    m = jnp.minimum(jnp.minimum(jnp.min(h1), jnp.min(h2)), jnp.min(h3))
    fill = m - 1.0
    xp_ref[pl.ds(0, N_NODES), pl.ds(0, D)] = h1
    xp_ref[pl.ds(0, N_NODES), pl.ds(D, D)] = h2
    xp_ref[pl.ds(0, N_NODES), pl.ds(2 * D, D)] = h3
    xp_ref[pl.ds(N_NODES, _N_PADDED - N_NODES), :] = jnp.full(
        (_N_PADDED - N_NODES, 3 * D), fill, jnp.float32)


def _tc_pad(h1, h2, h3):
    return pl.pallas_call(
        _pad_body,
        out_shape=jax.ShapeDtypeStruct((_N_PADDED, 3 * D), jnp.float32),
    )(h1, h2, h3)


def _mlp_body(xp_ref, w1_ref, b1_ref, w2_ref, b2_ref, out_ref):
    z = jnp.dot(xp_ref[...], w1_ref[...], preferred_element_type=jnp.float32,
                precision=_HIGH) + b1_ref[...]
    z = jnp.maximum(z, 0.0)
    z = jnp.dot(z, w2_ref[...], preferred_element_type=jnp.float32,
                precision=_HIGH) + b2_ref[...]
    mx = jnp.max(z, axis=-1, keepdims=True)
    lse = jnp.log(jnp.sum(jnp.exp(z - mx), axis=-1, keepdims=True)) + mx
    out_ref[...] = z - lse


def _tc_mlp(xp2, W1, b1_2d, W2, b2_2d):
    return pl.pallas_call(
        _mlp_body,
        out_shape=jax.ShapeDtypeStruct((NB, W2.shape[1]), jnp.float32),
    )(xp2, W1, b1_2d, W2, b2_2d)


@jax.jit
def kernel(x, edge_index, batch, edge_attr, Wc1, bc1, Wc2, bc2, Wc3, bc3,
           W1, b1, W2, b2):
    n = x.shape[0]
    loop_idx = jnp.arange(n, dtype=jnp.int32)
    pad = E_PAD - E_ALL
    src = jnp.concatenate([edge_index[0], loop_idx,
                           jnp.zeros((pad,), jnp.int32)])
    dst = jnp.concatenate([edge_index[1], loop_idx,
                           jnp.zeros((pad,), jnp.int32)])
    ew = jnp.concatenate([edge_attr, jnp.ones((n,), jnp.float32),
                          jnp.zeros((pad,), jnp.float32)])
    ewr = jnp.repeat(ew, 16)

    degp = _sc_deg(dst, ew)
    dis2d = _tc_dis(degp.reshape(2, DEG_ROWS, D))
    dis = dis2d.reshape(-1)[:N_NODES].reshape(N_NODES, 1)

    g1 = _tc_in(x, Wc1, dis)
    m1 = _sc_scatter(g1, src, dst, ewr)
    h1, g2 = _tc_mid(m1, bc1.reshape(1, -1), dis, Wc2)
    m2 = _sc_scatter(g2, src, dst, ewr)
    h2, g3 = _tc_mid(m2, bc2.reshape(1, -1), dis, Wc3)
    m3 = _sc_scatter(g3, src, dst, ewr)
    h3 = _tc_out(m3, bc3.reshape(1, -1), dis)

    xp = _tc_pad(h1, h2, h3)
    out = _tc_mlp(xp.reshape(NB, MAX_NODES * 3 * D), W1,
                  b1.reshape(1, -1), W2, b2.reshape(1, -1))
    return out
